# baseline (speedup 1.0000x reference)
import functools

import jax
import jax.numpy as jnp
from jax import lax
from jax.experimental import pallas as pl
from jax.experimental.pallas import tpu as pltpu
from jax.experimental.pallas import tpu_sc as plsc

VOCAB = 1000000
EMBED_DIM = 64
BATCH = 16384

_NC = 2
_NS = 16
_NW = _NC * _NS
_BPW = BATCH // _NW
_LANES = 16
_CHUNK = 128
_NCHUNK = _BPW // _CHUNK


def _sc_skipgram(target, context, table128):
    mesh = plsc.VectorSubcoreMesh(core_axis_name="c", subcore_axis_name="s")

    @functools.partial(
        pl.kernel,
        mesh=mesh,
        out_type=jax.ShapeDtypeStruct((BATCH,), jnp.float32),
        compiler_params=pltpu.CompilerParams(needs_layout_passes=False),
        scratch_types=[
            pltpu.VMEM((_BPW,), jnp.int32),
            pltpu.VMEM((_BPW,), jnp.int32),
            pltpu.VMEM((_BPW,), jnp.int32),
            pltpu.VMEM((_BPW,), jnp.int32),
            pltpu.VMEM((2, _CHUNK, 128), jnp.float32),
            pltpu.VMEM((2, _CHUNK, 128), jnp.float32),
            pltpu.VMEM((_BPW,), jnp.float32),
            pltpu.SemaphoreType.DMA,
            pltpu.SemaphoreType.DMA,
            pltpu.SemaphoreType.DMA,
            pltpu.SemaphoreType.DMA,
        ],
    )
    def k(tgt_hbm, ctx_hbm, table_hbm, out_hbm,
          idx_t, idx_c, idxh_t, idxh_c, rows_t, rows_c, scores,
          sem_t0, sem_t1, sem_c0, sem_c1):
        wid = lax.axis_index("s") * _NC + lax.axis_index("c")
        base = wid * _BPW

        pltpu.sync_copy(tgt_hbm.at[pl.ds(base, _BPW)], idx_t)
        pltpu.sync_copy(ctx_hbm.at[pl.ds(base, _BPW)], idx_c)

        def halve_body(i, _):
            sl = pl.ds(i * _LANES, _LANES)
            idxh_t[sl] = lax.shift_right_logical(idx_t[sl], 1)
            idxh_c[sl] = lax.shift_right_logical(idx_c[sl], 1)
            return 0

        lax.fori_loop(0, _BPW // _LANES, halve_body, 0)

        sems_t = (sem_t0, sem_t1)
        sems_c = (sem_c0, sem_c1)

        def start(g, slot):
            sl = pl.ds(g * _CHUNK, _CHUNK)
            cpt = pltpu.async_copy(table_hbm.at[idxh_t.at[sl]],
                                   rows_t.at[slot], sems_t[slot])
            cpc = pltpu.async_copy(table_hbm.at[idxh_c.at[sl]],
                                   rows_c.at[slot], sems_c[slot])
            return cpt, cpc

        lane = lax.iota(jnp.int32, _LANES)
        inflight = {0: start(0, 0)}

        for g in range(_NCHUNK):
            slot = g % 2
            if g + 1 < _NCHUNK:
                inflight[g + 1] = start(g + 1, (g + 1) % 2)
            cpt, cpc = inflight.pop(g)
            cpt.wait()
            cpc.wait()

            def chunk_body(i, _, slot=slot, g=g):
                vec = jnp.zeros((_LANES,), jnp.float32)
                gbase = g * _CHUNK + i * _LANES
                vts = idx_t[pl.ds(gbase, _LANES)]
                vcs = idx_c[pl.ds(gbase, _LANES)]
                for j in range(_LANES):
                    r = i * _LANES + j
                    pt = (vts[j] & 1) == 1
                    pc = (vcs[j] & 1) == 1
                    acc = jnp.zeros((_LANES,), jnp.float32)
                    for q in range(EMBED_DIM // _LANES):
                        tl = rows_t[slot, r, pl.ds(q * _LANES, _LANES)]
                        th = rows_t[slot, r, pl.ds(64 + q * _LANES, _LANES)]
                        cl = rows_c[slot, r, pl.ds(q * _LANES, _LANES)]
                        ch = rows_c[slot, r, pl.ds(64 + q * _LANES, _LANES)]
                        t = jnp.where(pt, th, tl)
                        c = jnp.where(pc, ch, cl)
                        acc = acc + t * c
                    vec = jnp.where(lane == j, jnp.sum(acc), vec)
                scores[pl.ds(gbase, _LANES)] = vec
                return 0

            lax.fori_loop(0, _CHUNK // _LANES, chunk_body, 0)

        pltpu.sync_copy(scores, out_hbm.at[pl.ds(base, _BPW)])

    return k(target, context, table128)


def kernel(target, context, emb_weight):
    table128 = jnp.reshape(emb_weight, (VOCAB // 2, 2 * EMBED_DIM))
    return _sc_skipgram(target.astype(jnp.int32), context.astype(jnp.int32),
                        table128)


# speedup vs baseline: 1.1186x; 1.1186x over previous
import functools

import jax
import jax.numpy as jnp
from jax import lax
from jax.experimental import pallas as pl
from jax.experimental.pallas import tpu as pltpu
from jax.experimental.pallas import tpu_sc as plsc

VOCAB = 1000000
EMBED_DIM = 64
BATCH = 16384

_NC = 2
_NS = 16
_NW = _NC * _NS
_BPW = BATCH // _NW
_LANES = 16
_CHUNK = 128
_NCHUNK = _BPW // _CHUNK


def _sc_skipgram(target, context, table128):
    mesh = plsc.VectorSubcoreMesh(core_axis_name="c", subcore_axis_name="s")

    @functools.partial(
        pl.kernel,
        mesh=mesh,
        out_type=jax.ShapeDtypeStruct((BATCH,), jnp.float32),
        compiler_params=pltpu.CompilerParams(needs_layout_passes=False),
        scratch_types=[
            pltpu.VMEM((_BPW,), jnp.int32),
            pltpu.VMEM((_BPW,), jnp.int32),
            pltpu.VMEM((2, _CHUNK, 128), jnp.float32),
            pltpu.VMEM((2, _CHUNK, 128), jnp.float32),
            pltpu.VMEM((_BPW,), jnp.float32),
            pltpu.SemaphoreType.DMA,
            pltpu.SemaphoreType.DMA,
            pltpu.SemaphoreType.DMA,
            pltpu.SemaphoreType.DMA,
        ],
    )
    def k(tgt_hbm, ctx_hbm, table_hbm, out_hbm,
          idx_t, idx_c, rows_t, rows_c, scores,
          sem_t0, sem_t1, sem_c0, sem_c1):
        wid = lax.axis_index("s") * _NC + lax.axis_index("c")
        base = wid * _BPW

        pltpu.sync_copy(tgt_hbm.at[pl.ds(base, _BPW)], idx_t)
        pltpu.sync_copy(ctx_hbm.at[pl.ds(base, _BPW)], idx_c)

        sems_t = (sem_t0, sem_t1)
        sems_c = (sem_c0, sem_c1)

        def start(g, slot):
            sl = pl.ds(g * _CHUNK, _CHUNK)
            cpt = pltpu.async_copy(table_hbm.at[idx_t.at[sl]],
                                   rows_t.at[slot], sems_t[slot])
            cpc = pltpu.async_copy(table_hbm.at[idx_c.at[sl]],
                                   rows_c.at[slot], sems_c[slot])
            return cpt, cpc

        lane = lax.iota(jnp.int32, _LANES)
        inflight = {0: start(0, 0)}

        for g in range(_NCHUNK):
            slot = g % 2
            if g + 1 < _NCHUNK:
                inflight[g + 1] = start(g + 1, (g + 1) % 2)
            cpt, cpc = inflight.pop(g)
            cpt.wait()
            cpc.wait()

            def chunk_body(i, _, slot=slot, g=g):
                vec = jnp.zeros((_LANES,), jnp.float32)
                for j in range(_LANES):
                    r = i * _LANES + j
                    acc = jnp.zeros((_LANES,), jnp.float32)
                    for q in range(EMBED_DIM // _LANES):
                        t = rows_t[slot, r, pl.ds(q * _LANES, _LANES)]
                        c = rows_c[slot, r, pl.ds(q * _LANES, _LANES)]
                        acc = acc + t * c
                    vec = jnp.where(lane == j, jnp.sum(acc), vec)
                scores[pl.ds(g * _CHUNK + i * _LANES, _LANES)] = vec
                return 0

            lax.fori_loop(0, _CHUNK // _LANES, chunk_body, 0)

        pltpu.sync_copy(scores, out_hbm.at[pl.ds(base, _BPW)])

    return k(target, context, table128)


def kernel(target, context, emb_weight):
    table128 = jnp.pad(emb_weight, ((0, 0), (0, 128 - EMBED_DIM)))
    return _sc_skipgram(target.astype(jnp.int32), context.astype(jnp.int32),
                        table128)
